# baseline (speedup 1.0000x reference)
import jax
import jax.numpy as jnp
from jax import lax
from jax.experimental import pallas as pl
from jax.experimental.pallas import tpu as pltpu
from jax.experimental.pallas import tpu_sc as plsc

NUM_CLASSES = 100000
EMBED_DIM = 64
BATCH = 16384

_INFO = plsc.get_sparse_core_info()
_NC = _INFO.num_cores
_NS = _INFO.num_subcores
_NW = _NC * _NS
_B_PER_W = BATCH // _NW


_CHUNK = 64
_CHUNKS = _B_PER_W // _CHUNK


def _emb_body(idx_hbm, table_hbm, out_hbm, idx_v, rows_v, sem_g, sem_w):
    wid = lax.axis_index("s") * _NC + lax.axis_index("c")
    base = wid * _B_PER_W
    pltpu.sync_copy(idx_hbm.at[pl.ds(base, _B_PER_W)], idx_v)
    gathers = []
    for c in range(_CHUNKS):
        gathers.append(pltpu.async_copy(
            table_hbm.at[idx_v.at[pl.ds(c * _CHUNK, _CHUNK)]],
            rows_v.at[pl.ds(c * _CHUNK, _CHUNK)], sem_g.at[c]))
    writes = []
    for c in range(_CHUNKS):
        gathers[c].wait()
        writes.append(pltpu.async_copy(
            rows_v.at[pl.ds(c * _CHUNK, _CHUNK)],
            out_hbm.at[pl.ds(base + c * _CHUNK, _CHUNK)], sem_w.at[c]))
    for w in writes:
        w.wait()


@jax.jit
def _emb(class_labels, table):
    mesh = plsc.VectorSubcoreMesh(core_axis_name="c", subcore_axis_name="s")
    return pl.kernel(
        _emb_body,
        mesh=mesh,
        out_type=jax.ShapeDtypeStruct((BATCH, EMBED_DIM), jnp.float32),
        scratch_types=[
            pltpu.VMEM((_B_PER_W,), jnp.int32),
            pltpu.VMEM((_B_PER_W, EMBED_DIM), jnp.float32),
            pltpu.SemaphoreType.DMA((_CHUNKS,)),
            pltpu.SemaphoreType.DMA((_CHUNKS,)),
        ],
        compiler_params=pltpu.CompilerParams(use_tc_tiling_on_sc=False),
    )(class_labels, table)


def kernel(class_labels, table):
    return _emb(class_labels.astype(jnp.int32), table)


# speedup vs baseline: 1.9861x; 1.9861x over previous
import jax
import jax.numpy as jnp
from jax import lax
from jax.experimental import pallas as pl
from jax.experimental.pallas import tpu as pltpu
from jax.experimental.pallas import tpu_sc as plsc

NUM_CLASSES = 100000
EMBED_DIM = 64
BATCH = 16384

_INFO = plsc.get_sparse_core_info()
_NC = _INFO.num_cores
_NS = _INFO.num_subcores
_NW = _NC * _NS
_DIMS_PER_W = EMBED_DIM // _NW
_HALF = BATCH // 2
_V = NUM_CLASSES + 1


def _emb_body(idx_hbm, tab_hbm, out_hbm, idx_v, row_v, out_v):
    wid = lax.axis_index("s") * _NC + lax.axis_index("c")
    for d in range(_DIMS_PER_W):
        j = wid * _DIMS_PER_W + d
        pltpu.sync_copy(tab_hbm.at[j], row_v)
        for h in range(2):
            pltpu.sync_copy(idx_hbm.at[pl.ds(h * _HALF, _HALF)], idx_v)

            def gbody(i, _):
                idxs = idx_v[pl.ds(i * 16, 16)]
                out_v[pl.ds(i * 16, 16)] = plsc.load_gather(row_v, [idxs])
                return 0

            lax.fori_loop(0, _HALF // 16, gbody, 0)
            pltpu.sync_copy(out_v, out_hbm.at[j, pl.ds(h * _HALF, _HALF)])


@jax.jit
def _emb(class_labels, table_t):
    mesh = plsc.VectorSubcoreMesh(core_axis_name="c", subcore_axis_name="s")
    return pl.kernel(
        _emb_body,
        mesh=mesh,
        out_type=jax.ShapeDtypeStruct((EMBED_DIM, BATCH), jnp.float32),
        scratch_types=[
            pltpu.VMEM((_HALF,), jnp.int32),
            pltpu.VMEM((_V,), jnp.float32),
            pltpu.VMEM((_HALF,), jnp.float32),
        ],
        compiler_params=pltpu.CompilerParams(
            use_tc_tiling_on_sc=True, needs_layout_passes=False),
    )(class_labels, table_t)


def kernel(class_labels, table):
    out_t = _emb(class_labels.astype(jnp.int32), table.T)
    return out_t.T


# speedup vs baseline: 2.2752x; 1.1456x over previous
import jax
import jax.numpy as jnp
from jax import lax
from jax.experimental import pallas as pl
from jax.experimental.pallas import tpu as pltpu
from jax.experimental.pallas import tpu_sc as plsc

NUM_CLASSES = 100000
EMBED_DIM = 64
BATCH = 16384

_INFO = plsc.get_sparse_core_info()
_NC = _INFO.num_cores
_NS = _INFO.num_subcores
_NW = _NC * _NS
_DIMS_PER_W = EMBED_DIM // _NW
_Q = BATCH // 4
_V = NUM_CLASSES + 1
_UNROLL = 8


def _emb_body(idx_hbm, tab_hbm, out_hbm, idx_a, idx_b, row_v, out_a, out_b,
              sem_r, sem_ia, sem_ib, sem_oa, sem_ob):
    wid = lax.axis_index("s") * _NC + lax.axis_index("c")
    idx_bufs = (idx_a, idx_b)
    idx_sems = (sem_ia, sem_ib)
    out_bufs = (out_a, out_b)
    out_sems = (sem_oa, sem_ob)

    def gather_quarter(src_idx, dst_out):
        def gbody(i, _):
            base = i * (16 * _UNROLL)
            for u in range(_UNROLL):
                s = pl.ds(base + u * 16, 16)
                dst_out[s] = plsc.load_gather(row_v, [src_idx[s]])
            return 0
        lax.fori_loop(0, _Q // (16 * _UNROLL), gbody, 0)

    out_waits = [None, None, None, None]
    for d in range(_DIMS_PER_W):
        j = wid * _DIMS_PER_W + d
        row_cp = pltpu.async_copy(tab_hbm.at[j], row_v, sem_r)
        idx_cp = pltpu.async_copy(idx_hbm.at[pl.ds(0, _Q)], idx_bufs[0], sem_ia)
        row_cp.wait()
        for q in range(4):
            b = q % 2
            nb = (q + 1) % 2
            idx_cp.wait()
            if q < 3:
                idx_cp = pltpu.async_copy(
                    idx_hbm.at[pl.ds((q + 1) * _Q, _Q)], idx_bufs[nb],
                    idx_sems[nb])
            if out_waits[b] is not None:
                out_waits[b].wait()
            gather_quarter(idx_bufs[b], out_bufs[b])
            out_waits[b] = pltpu.async_copy(
                out_bufs[b], out_hbm.at[j, pl.ds(q * _Q, _Q)], out_sems[b])
    out_waits[0].wait()
    out_waits[1].wait()


@jax.jit
def _emb(class_labels, table_t):
    mesh = plsc.VectorSubcoreMesh(core_axis_name="c", subcore_axis_name="s")
    return pl.kernel(
        _emb_body,
        mesh=mesh,
        out_type=jax.ShapeDtypeStruct((EMBED_DIM, BATCH), jnp.float32),
        scratch_types=[
            pltpu.VMEM((_Q,), jnp.int32),
            pltpu.VMEM((_Q,), jnp.int32),
            pltpu.VMEM((_V,), jnp.float32),
            pltpu.VMEM((_Q,), jnp.float32),
            pltpu.VMEM((_Q,), jnp.float32),
            pltpu.SemaphoreType.DMA,
            pltpu.SemaphoreType.DMA,
            pltpu.SemaphoreType.DMA,
            pltpu.SemaphoreType.DMA,
            pltpu.SemaphoreType.DMA,
        ],
        compiler_params=pltpu.CompilerParams(
            use_tc_tiling_on_sc=True, needs_layout_passes=False),
    )(class_labels, table_t)


def kernel(class_labels, table):
    out_t = _emb(class_labels.astype(jnp.int32), table.T)
    return out_t.T


# speedup vs baseline: 2.3757x; 1.0441x over previous
import jax
import jax.numpy as jnp
from jax import lax
from jax.experimental import pallas as pl
from jax.experimental.pallas import tpu as pltpu
from jax.experimental.pallas import tpu_sc as plsc

NUM_CLASSES = 100000
EMBED_DIM = 64
BATCH = 16384

_INFO = plsc.get_sparse_core_info()
_NC = _INFO.num_cores
_NS = _INFO.num_subcores
_NW = _NC * _NS
_DIMS_PER_W = EMBED_DIM // _NW
_Q = BATCH // 4
_V = NUM_CLASSES + 1
_UNROLL = 8


def _emb_body(idx_hbm, tab_hbm, out_hbm, idx_v, row_v, out_a, out_b,
              sem_r, sem_i, sem_oa, sem_ob):
    wid = lax.axis_index("s") * _NC + lax.axis_index("c")
    out_bufs = (out_a, out_b)
    out_sems = (sem_oa, sem_ob)

    def gather_quarter(q, dst_out):
        def gbody(i, _):
            base = q * _Q + i * (16 * _UNROLL)
            for u in range(_UNROLL):
                s = pl.ds(base + u * 16, 16)
                dst_out[pl.ds(i * (16 * _UNROLL) + u * 16, 16)] = (
                    plsc.load_gather(row_v, [idx_v[s]]))
            return 0
        lax.fori_loop(0, _Q // (16 * _UNROLL), gbody, 0)

    idx_cp = pltpu.async_copy(idx_hbm, idx_v, sem_i)
    out_waits = [None, None]
    idx_waited = False
    for d in range(_DIMS_PER_W):
        j = wid * _DIMS_PER_W + d
        row_cp = pltpu.async_copy(tab_hbm.at[j], row_v, sem_r)
        row_cp.wait()
        if not idx_waited:
            idx_cp.wait()
            idx_waited = True
        for q in range(4):
            b = q % 2
            if out_waits[b] is not None:
                out_waits[b].wait()
            gather_quarter(q, out_bufs[b])
            out_waits[b] = pltpu.async_copy(
                out_bufs[b], out_hbm.at[j, pl.ds(q * _Q, _Q)], out_sems[b])
    out_waits[0].wait()
    out_waits[1].wait()


@jax.jit
def _emb(class_labels, table_t):
    mesh = plsc.VectorSubcoreMesh(core_axis_name="c", subcore_axis_name="s")
    return pl.kernel(
        _emb_body,
        mesh=mesh,
        out_type=jax.ShapeDtypeStruct((EMBED_DIM, BATCH), jnp.float32),
        scratch_types=[
            pltpu.VMEM((BATCH,), jnp.int32),
            pltpu.VMEM((_V,), jnp.float32),
            pltpu.VMEM((_Q,), jnp.float32),
            pltpu.VMEM((_Q,), jnp.float32),
            pltpu.SemaphoreType.DMA,
            pltpu.SemaphoreType.DMA,
            pltpu.SemaphoreType.DMA,
            pltpu.SemaphoreType.DMA,
        ],
        compiler_params=pltpu.CompilerParams(
            use_tc_tiling_on_sc=True, needs_layout_passes=False),
    )(class_labels, table_t)


def kernel(class_labels, table):
    out_t = _emb(class_labels.astype(jnp.int32), table.T)
    return out_t.T


# speedup vs baseline: 2.5979x; 1.0936x over previous
import jax
import jax.numpy as jnp
from jax import lax
from jax.experimental import pallas as pl
from jax.experimental.pallas import tpu as pltpu
from jax.experimental.pallas import tpu_sc as plsc

NUM_CLASSES = 100000
EMBED_DIM = 64
BATCH = 16384

_INFO = plsc.get_sparse_core_info()
_NC = _INFO.num_cores
_NS = _INFO.num_subcores
_NW = _NC * _NS
_DIMS_PER_W = EMBED_DIM // _NW
_Q = BATCH // 4
_V = NUM_CLASSES + 1
_UNROLL = 8


def _emb_body(idx_hbm, tab_hbm, out_hbm, idx_v, row_v, out_a, out_b,
              sem_r, sem_i, sem_oa, sem_ob):
    wid = lax.axis_index("s") * _NC + lax.axis_index("c")
    out_bufs = (out_a, out_b)
    out_sems = (sem_oa, sem_ob)

    def gather_quarter(q, dst_out):
        @plsc.parallel_loop(0, _Q, step=16 * _UNROLL)
        def gbody(i):
            for u in range(_UNROLL):
                s = pl.ds(q * _Q + i + u * 16, 16)
                dst_out[pl.ds(i + u * 16, 16)] = (
                    plsc.load_gather(row_v, [idx_v[s]]))

    idx_cp = pltpu.async_copy(idx_hbm, idx_v, sem_i)
    out_waits = [None, None]
    idx_waited = False
    for d in range(_DIMS_PER_W):
        j = wid * _DIMS_PER_W + d
        row_cp = pltpu.async_copy(tab_hbm.at[j], row_v, sem_r)
        row_cp.wait()
        if not idx_waited:
            idx_cp.wait()
            idx_waited = True
        for q in range(4):
            b = q % 2
            if out_waits[b] is not None:
                out_waits[b].wait()
            gather_quarter(q, out_bufs[b])
            out_waits[b] = pltpu.async_copy(
                out_bufs[b], out_hbm.at[j, pl.ds(q * _Q, _Q)], out_sems[b])
    out_waits[0].wait()
    out_waits[1].wait()


@jax.jit
def _emb(class_labels, table_t):
    mesh = plsc.VectorSubcoreMesh(core_axis_name="c", subcore_axis_name="s")
    return pl.kernel(
        _emb_body,
        mesh=mesh,
        out_type=jax.ShapeDtypeStruct((EMBED_DIM, BATCH), jnp.float32),
        scratch_types=[
            pltpu.VMEM((BATCH,), jnp.int32),
            pltpu.VMEM((_V,), jnp.float32),
            pltpu.VMEM((_Q,), jnp.float32),
            pltpu.VMEM((_Q,), jnp.float32),
            pltpu.SemaphoreType.DMA,
            pltpu.SemaphoreType.DMA,
            pltpu.SemaphoreType.DMA,
            pltpu.SemaphoreType.DMA,
        ],
        compiler_params=pltpu.CompilerParams(
            use_tc_tiling_on_sc=True, needs_layout_passes=False),
    )(class_labels, table_t)


def kernel(class_labels, table):
    out_t = _emb(class_labels.astype(jnp.int32), table.T)
    return out_t.T
